# baseline (speedup 1.0000x reference)
import functools

import jax
import jax.numpy as jnp
import numpy as np
from jax.experimental import pallas as pl

NUM_CHANNEL = 4
NGRAM_SIZE = 3
LEVELS = 100
DIM = 10000
SEQ_LEN = 256
CHOSEN_FEAT = [547, 548, 549, 551, 554, 556, 557, 558, 559, 560, 561, 562,
               563, 565, 566, 567, 570, 576, 580, 581, 582, 583, 584, 585,
               588, 593, 598, 599, 600]

BLK = 2048
WEXT = BLK + 128
NBLK = pl.cdiv(DIM, BLK)
PADW = (NBLK - 1) * BLK + WEXT
CPAD = 128


def _encoder_kernel(sig_ref, effpad_ref, vals_ref, wf_ref,
                    bf_ref, mf_ref, wm_ref, bm_ref, out_ref):
    i = pl.program_id(0)
    d0 = i * BLK

    eff_win = effpad_ref[:, pl.ds(d0, WEXT)]
    sig = sig_ref[...]
    idx = jnp.clip(jnp.floor(sig * LEVELS).astype(jnp.int32), 0, LEVELS - 1)
    iota = jax.lax.broadcasted_iota(jnp.int32, (SEQ_LEN, NUM_CHANNEL * CPAD), 1)
    ohm = (iota == idx[:, 0][:, None])
    for c in range(1, NUM_CHANNEL):
        ohm = ohm | (iota == idx[:, c][:, None] + c * CPAD)
    oh = ohm.astype(jnp.bfloat16)
    ch = jax.lax.dot_general(
        oh, eff_win, (((1,), (0,)), ((), ())),
        preferred_element_type=jnp.float32).astype(jnp.bfloat16)

    tt = SEQ_LEN - NGRAM_SIZE + 1
    a = ch[0:tt, 0:BLK]
    b = ch[1:tt + 1, 1:BLK + 1]
    cc = ch[2:tt + 2, 2:BLK + 2]
    grams = a * b * cc
    ones = jnp.ones((1, tt), jnp.bfloat16)
    sample_hv = jax.lax.dot_general(
        ones, grams, (((1,), (0,)), ((), ())),
        preferred_element_type=jnp.float32)[0]

    proj = vals_ref[...] * wf_ref[...]
    fhv = jnp.cos(proj + bf_ref[...]) * jnp.sin(proj)

    wm = wm_ref[...].astype(jnp.bfloat16)
    mf = mf_ref[...].astype(jnp.bfloat16)
    mproj = jax.lax.dot_general(mf, wm, (((1,), (2,)), ((0,), (0,))),
                                preferred_element_type=jnp.float32)
    mhv = jnp.cos(mproj + bm_ref[...]) * jnp.sin(mproj)
    mfcc_hv = mhv[0] * mhv[1] * mhv[2] * mhv[3] * mhv[4] * mhv[5]

    f = {cf: fhv[j] for j, cf in enumerate(CHOSEN_FEAT)}
    expr = (f[547] * f[559] * f[565]
            + f[548] * f[560] * f[566]
            + f[549] * f[561] * f[567]
            + f[551] * f[554]
            + f[556] * f[558] * f[584] * f[557] * f[585] * f[581] * f[580]
            * f[582] * f[583] * f[598] * f[600] * f[599]
            + f[562] + f[563]
            + f[570] * f[588]
            + f[576] + f[593]
            + mfcc_hv)

    out = sample_hv * expr
    out_ref[0, :] = jnp.where(out > 0, 1.0, -1.0)


@functools.partial(jax.jit, static_argnames=("interpret",))
def _run(signals, feat, keys, level_table, W_feat, b_feat, W_mfcc, b_mfcc,
         interpret=False):
    eff = (keys.astype(jnp.bfloat16)[:, None, :]
           * level_table.astype(jnp.bfloat16)[None, :, :])
    eff = jnp.pad(eff, ((0, 0), (0, CPAD - LEVELS), (0, 0)))
    eff = eff.reshape(NUM_CHANNEL * CPAD, DIM)
    effpad = jnp.concatenate(
        [eff[:, -2:], eff,
         jnp.zeros((NUM_CHANNEL * CPAD, PADW - DIM - 2), eff.dtype)], axis=-1)
    sel = np.array([cf - 1 for cf in CHOSEN_FEAT])
    vals = feat[sel][:, None]
    mf = feat[: 6 * 91].reshape(6, 91)

    out = pl.pallas_call(
        _encoder_kernel,
        grid=(NBLK,),
        in_specs=[
            pl.BlockSpec((SEQ_LEN, NUM_CHANNEL), lambda i: (0, 0)),
            pl.BlockSpec((NUM_CHANNEL * CPAD, PADW), lambda i: (0, 0)),
            pl.BlockSpec((len(CHOSEN_FEAT), 1), lambda i: (0, 0)),
            pl.BlockSpec((len(CHOSEN_FEAT), BLK), lambda i: (0, i)),
            pl.BlockSpec((len(CHOSEN_FEAT), BLK), lambda i: (0, i)),
            pl.BlockSpec((6, 91), lambda i: (0, 0)),
            pl.BlockSpec((6, BLK, 91), lambda i: (0, i, 0)),
            pl.BlockSpec((6, BLK), lambda i: (0, i)),
        ],
        out_specs=pl.BlockSpec((1, BLK), lambda i: (0, i)),
        out_shape=jax.ShapeDtypeStruct((1, DIM), jnp.float32),
        interpret=interpret,
    )(signals, effpad, vals, W_feat, b_feat, mf, W_mfcc, b_mfcc)
    return out.reshape(-1)


def kernel(signals, feat, keys, level_table, W_feat, b_feat, W_mfcc, b_mfcc):
    return _run(signals, feat, keys, level_table, W_feat, b_feat,
                W_mfcc, b_mfcc)
